# baseline (speedup 1.0000x reference)
import functools

import jax
import jax.numpy as jnp
from jax.experimental import pallas as pl
from jax.experimental.pallas import tpu as pltpu

_BR = 256


def _nce_body(inv_ref, lab_a, log_a, lab_b, log_b, out_ref):
    inv = inv_ref[0]

    def half(lab_ref, log_ref):
        lab = lab_ref[:, :]
        lo = log_ref[:, :]
        m = jnp.max(lab, axis=1, keepdims=True)
        pos = jnp.max(jnp.where(lab == m, lo, -jnp.inf), axis=1)
        rm = jnp.max(lo, axis=1, keepdims=True)
        se = jnp.sum(jnp.exp((lo - rm) * inv), axis=1)
        return jnp.sum(jnp.log(se) + (rm[:, 0] - pos) * inv)

    block_sum = half(lab_a, log_a) + half(lab_b, log_b)

    @pl.when(pl.program_id(0) == 0)
    def _init():
        out_ref[0, 0] = 0.0

    out_ref[0, 0] += block_sum


@functools.partial(jax.jit, static_argnames=())
def kernel(labels, logits, mask, alpha):
    del mask
    n, d = logits.shape
    inv = (1.0 / alpha) * jnp.ones((1,), dtype=jnp.float32)
    grid = n // (2 * _BR)
    out = pl.pallas_call(
        _nce_body,
        grid=(grid,),
        in_specs=[
            pl.BlockSpec(memory_space=pltpu.SMEM),
            pl.BlockSpec((_BR, d), lambda i: (2 * i, 0)),
            pl.BlockSpec((_BR, d), lambda i: (2 * i, 0)),
            pl.BlockSpec((_BR, d), lambda i: (2 * i + 1, 0)),
            pl.BlockSpec((_BR, d), lambda i: (2 * i + 1, 0)),
        ],
        out_specs=pl.BlockSpec(memory_space=pltpu.SMEM),
        out_shape=jax.ShapeDtypeStruct((1, 1), jnp.float32),
    )(inv, labels, logits, labels, logits)
    return out[0, 0] / n


# speedup vs baseline: 1.0052x; 1.0052x over previous
import functools

import jax
import jax.numpy as jnp
from jax.experimental import pallas as pl
from jax.experimental.pallas import tpu as pltpu

_BR = 512


def _nce_body(inv_ref, lab_ref, log_ref, out_ref):
    inv = inv_ref[0]
    lab = lab_ref[:, :]
    lo = log_ref[:, :]
    m = jnp.max(lab, axis=1, keepdims=True)
    pos = jnp.max(jnp.where(lab == m, lo, -jnp.inf), axis=1)
    rm = jnp.max(lo, axis=1, keepdims=True)
    se = jnp.sum(jnp.exp((lo - rm) * inv), axis=1)
    block_sum = jnp.sum(jnp.log(se) + (rm[:, 0] - pos) * inv)

    @pl.when(pl.program_id(0) == 0)
    def _init():
        out_ref[0, 0] = 0.0

    out_ref[0, 0] += block_sum


@functools.partial(jax.jit, static_argnames=())
def kernel(labels, logits, mask, alpha):
    del mask
    n, d = logits.shape
    inv = (1.0 / alpha) * jnp.ones((1,), dtype=jnp.float32)
    grid = n // _BR
    out = pl.pallas_call(
        _nce_body,
        grid=(grid,),
        in_specs=[
            pl.BlockSpec(memory_space=pltpu.SMEM),
            pl.BlockSpec((_BR, d), lambda i: (i, 0)),
            pl.BlockSpec((_BR, d), lambda i: (i, 0)),
        ],
        out_specs=pl.BlockSpec(memory_space=pltpu.SMEM),
        out_shape=jax.ShapeDtypeStruct((1, 1), jnp.float32),
    )(inv, labels, logits)
    return out[0, 0] / n
